# Initial kernel scaffold; baseline (speedup 1.0000x reference)
import jax
import jax.numpy as jnp
from jax import lax
from jax.experimental import pallas as pl
from jax.experimental.pallas import tpu as pltpu
from jax.experimental.pallas import tpu_sc as plsc

B = 16384
W = 20
NEG = 20
D = 64

NC = 2
NS = 16
NW = NC * NS

EPT = B // NW
CHUNK = 16
NCHUNK = EPT // CHUNK
IPC = CHUNK * W
GSPLIT = 4
GSZ = IPC // GSPLIT


def _sc_body(pos_u_h, pos_v_h, neg_v_h, pos_d_h, u_emb_h, d_emb_h, v_emb_h,
             pos_out_h, neg_out_h,
             u_idx, n_idx, v_idx, d_idx,
             u_rows, n_rows, v_rows, d_rows,
             dot_buf, pos_all, neg_all, sem):
    wid = lax.axis_index("s") * NC + lax.axis_index("c")
    base_c = wid * NCHUNK

    @pl.loop(0, NCHUNK)
    def _chunks(c):
        cid = base_c + c
        eb = pl.multiple_of(cid * CHUNK, CHUNK)
        pltpu.sync_copy(pos_u_h.at[cid], u_idx)
        pltpu.sync_copy(neg_v_h.at[cid], n_idx)
        pltpu.sync_copy(pos_v_h.at[pl.ds(eb, CHUNK)], v_idx)
        pltpu.sync_copy(pos_d_h.at[pl.ds(eb, CHUNK)], d_idx)
        cps = []
        for j in range(GSPLIT):
            cps.append(pltpu.async_copy(
                u_emb_h.at[u_idx.at[j]], u_rows.at[pl.ds(j * GSZ, GSZ)], sem))
            cps.append(pltpu.async_copy(
                v_emb_h.at[n_idx.at[j]], n_rows.at[pl.ds(j * GSZ, GSZ)], sem))
        cps.append(pltpu.async_copy(v_emb_h.at[v_idx], v_rows, sem))
        cps.append(pltpu.async_copy(d_emb_h.at[d_idx], d_rows, sem))
        for cp in cps:
            cp.wait()

        @pl.loop(0, CHUNK)
        def _elems(e):
            rb = e * W
            acc = [d_rows[e, pl.ds(k * 16, 16)] for k in range(4)]
            for w in range(W):
                for k in range(4):
                    acc[k] = acc[k] + u_rows[rb + w, pl.ds(k * 16, 16)]
            scale = 1.0 / (W + 1)
            acc = [a * scale for a in acc]
            pt = v_rows[e, pl.ds(0, 16)] * acc[0]
            for k in range(1, 4):
                pt = pt + v_rows[e, pl.ds(k * 16, 16)] * acc[k]
            pos_all[c * CHUNK + e] = jnp.sum(pt)
            for n in range(NEG):
                t = n_rows[rb + n, pl.ds(0, 16)] * acc[0]
                for k in range(1, 4):
                    t = t + n_rows[rb + n, pl.ds(k * 16, 16)] * acc[k]
                dot_buf[n, e] = jnp.sum(t)

        ns = jnp.zeros((16,), jnp.float32)
        for n in range(NEG):
            ns = ns + jnp.exp(jnp.minimum(dot_buf[n, :], 50.0))
        neg_all[pl.ds(c * CHUNK, CHUNK)] = ns

    pltpu.sync_copy(pos_all, pos_out_h.at[pl.ds(wid * EPT, EPT)])
    pltpu.sync_copy(neg_all, neg_out_h.at[pl.ds(wid * EPT, EPT)])


@jax.jit
def _sc_gather_scores(pos_u_r, pos_v, neg_v_r, pos_d, u_emb, d_emb, v_emb):
    mesh = plsc.VectorSubcoreMesh(
        core_axis_name="c", subcore_axis_name="s",
        num_cores=NC, num_subcores=NS)
    f = pl.kernel(
        _sc_body,
        out_type=[jax.ShapeDtypeStruct((B,), jnp.float32),
                  jax.ShapeDtypeStruct((B,), jnp.float32)],
        mesh=mesh,
        scratch_types=[
            pltpu.VMEM((GSPLIT, GSZ), jnp.int32),
            pltpu.VMEM((GSPLIT, GSZ), jnp.int32),
            pltpu.VMEM((CHUNK,), jnp.int32),
            pltpu.VMEM((CHUNK,), jnp.int32),
            pltpu.VMEM((IPC, D), jnp.float32),
            pltpu.VMEM((IPC, D), jnp.float32),
            pltpu.VMEM((CHUNK, D), jnp.float32),
            pltpu.VMEM((CHUNK, D), jnp.float32),
            pltpu.VMEM((NEG, CHUNK), jnp.float32),
            pltpu.VMEM((EPT,), jnp.float32),
            pltpu.VMEM((EPT,), jnp.float32),
            pltpu.SemaphoreType.DMA,
        ],
    )
    return f(pos_u_r, pos_v, neg_v_r, pos_d, u_emb, d_emb, v_emb)


def _loss_body(pos_ref, neg_ref, out_ref):
    p = jnp.minimum(pos_ref[...], 50.0)
    s = p - jnp.log(jnp.exp(p) + neg_ref[...])
    out_ref[0, 0] = -jnp.sum(s) / B


def kernel(pos_u, pos_v, neg_v, pos_d, u_emb, d_emb, v_emb):
    pos_u_r = pos_u.astype(jnp.int32).reshape(B * W // IPC, GSPLIT, GSZ)
    neg_v_r = neg_v.astype(jnp.int32).reshape(B * NEG // IPC, GSPLIT, GSZ)
    pos, neg = _sc_gather_scores(
        pos_u_r, pos_v.astype(jnp.int32), neg_v_r, pos_d.astype(jnp.int32),
        u_emb, d_emb, v_emb)
    loss = pl.pallas_call(
        _loss_body,
        out_shape=jax.ShapeDtypeStruct((1, 1), jnp.float32),
        out_specs=pl.BlockSpec(memory_space=pltpu.SMEM),
    )(pos.reshape(128, 128), neg.reshape(128, 128))
    return loss.reshape(())


# speedup vs baseline: 4.9619x; 4.9619x over previous
import jax
import jax.numpy as jnp
from jax import lax
from jax.experimental import pallas as pl
from jax.experimental.pallas import tpu as pltpu
from jax.experimental.pallas import tpu_sc as plsc

B = 16384
W = 20
NEG = 20
D = 64

NC = 2
NS = 16
NW = NC * NS

EPT = B // NW
CHUNK = 16
NCHUNK = EPT // CHUNK
IPC = CHUNK * W
GSPLIT = 4
GSZ = IPC // GSPLIT


def _sc_body(pos_u_h, pos_v_h, neg_v_h, pos_d_h, u_emb_h, d_emb_h, v_emb_h,
             pos_out_h, neg_out_h,
             u_idx, n_idx, v_idx, d_idx,
             u_rows, n_rows, v_rows, d_rows,
             pos_all, neg_all, sem):
    wid = lax.axis_index("s") * NC + lax.axis_index("c")
    base_c = wid * NCHUNK

    @pl.loop(0, NCHUNK)
    def _chunks(c):
        cid = base_c + c
        eb = pl.multiple_of(cid * CHUNK, CHUNK)
        pltpu.sync_copy(pos_u_h.at[cid], u_idx)
        pltpu.sync_copy(neg_v_h.at[cid], n_idx)
        pltpu.sync_copy(pos_v_h.at[pl.ds(eb, CHUNK)], v_idx)
        pltpu.sync_copy(pos_d_h.at[pl.ds(eb, CHUNK)], d_idx)
        cps = []
        for j in range(GSPLIT):
            cps.append(pltpu.async_copy(
                u_emb_h.at[u_idx.at[j]], u_rows.at[pl.ds(j * GSZ, GSZ)], sem))
            cps.append(pltpu.async_copy(
                v_emb_h.at[n_idx.at[j]], n_rows.at[pl.ds(j * GSZ, GSZ)], sem))
        cps.append(pltpu.async_copy(v_emb_h.at[v_idx], v_rows, sem))
        cps.append(pltpu.async_copy(d_emb_h.at[d_idx], d_rows, sem))
        for cp in cps:
            cp.wait()

        iota = lax.iota(jnp.int32, 16)
        zero = jnp.zeros((16,), jnp.float32)

        @pl.loop(0, CHUNK, init_carry=(zero, [zero] * NEG))
        def _elems(e, carry):
            pos_acc, negs = carry
            rb = e * W
            acc = [d_rows[e, pl.ds(k * 16, 16)] for k in range(4)]
            for w in range(W):
                for k in range(4):
                    acc[k] = acc[k] + u_rows[rb + w, pl.ds(k * 16, 16)]
            acc = [a * (1.0 / (W + 1)) for a in acc]
            pt = v_rows[e, pl.ds(0, 16)] * acc[0]
            for k in range(1, 4):
                pt = pt + v_rows[e, pl.ds(k * 16, 16)] * acc[k]
            lane = iota == e
            pos_acc = jnp.where(lane, jnp.sum(pt), pos_acc)
            out_negs = []
            for n in range(NEG):
                t = n_rows[rb + n, pl.ds(0, 16)] * acc[0]
                for k in range(1, 4):
                    t = t + n_rows[rb + n, pl.ds(k * 16, 16)] * acc[k]
                out_negs.append(jnp.where(lane, jnp.sum(t), negs[n]))
            return pos_acc, out_negs

        pos, negs = _elems
        ns = jnp.zeros((16,), jnp.float32)
        for n in range(NEG):
            ns = ns + jnp.exp(jnp.minimum(negs[n], 50.0))
        pos_all[pl.ds(c * CHUNK, CHUNK)] = pos
        neg_all[pl.ds(c * CHUNK, CHUNK)] = ns

    pltpu.sync_copy(pos_all, pos_out_h.at[pl.ds(wid * EPT, EPT)])
    pltpu.sync_copy(neg_all, neg_out_h.at[pl.ds(wid * EPT, EPT)])


@jax.jit
def _sc_gather_scores(pos_u_r, pos_v, neg_v_r, pos_d, u_emb, d_emb, v_emb):
    mesh = plsc.VectorSubcoreMesh(
        core_axis_name="c", subcore_axis_name="s",
        num_cores=NC, num_subcores=NS)
    f = pl.kernel(
        _sc_body,
        out_type=[jax.ShapeDtypeStruct((B,), jnp.float32),
                  jax.ShapeDtypeStruct((B,), jnp.float32)],
        mesh=mesh,
        compiler_params=pltpu.CompilerParams(
            needs_layout_passes=False, use_tc_tiling_on_sc=False),
        scratch_types=[
            pltpu.VMEM((GSPLIT, GSZ), jnp.int32),
            pltpu.VMEM((GSPLIT, GSZ), jnp.int32),
            pltpu.VMEM((CHUNK,), jnp.int32),
            pltpu.VMEM((CHUNK,), jnp.int32),
            pltpu.VMEM((IPC, D), jnp.float32),
            pltpu.VMEM((IPC, D), jnp.float32),
            pltpu.VMEM((CHUNK, D), jnp.float32),
            pltpu.VMEM((CHUNK, D), jnp.float32),
            pltpu.VMEM((EPT,), jnp.float32),
            pltpu.VMEM((EPT,), jnp.float32),
            pltpu.SemaphoreType.DMA,
        ],
    )
    return f(pos_u_r, pos_v, neg_v_r, pos_d, u_emb, d_emb, v_emb)


def _loss_body(pos_ref, neg_ref, out_ref):
    p = jnp.minimum(pos_ref[...], 50.0)
    s = p - jnp.log(jnp.exp(p) + neg_ref[...])
    out_ref[0, 0] = -jnp.sum(s) / B


def kernel(pos_u, pos_v, neg_v, pos_d, u_emb, d_emb, v_emb):
    pos_u_r = pos_u.astype(jnp.int32).reshape(B * W // IPC, GSPLIT, GSZ)
    neg_v_r = neg_v.astype(jnp.int32).reshape(B * NEG // IPC, GSPLIT, GSZ)
    pos, neg = _sc_gather_scores(
        pos_u_r, pos_v.astype(jnp.int32), neg_v_r, pos_d.astype(jnp.int32),
        u_emb, d_emb, v_emb)
    loss = pl.pallas_call(
        _loss_body,
        out_shape=jax.ShapeDtypeStruct((1, 1), jnp.float32),
        out_specs=pl.BlockSpec(memory_space=pltpu.SMEM),
    )(pos.reshape(128, 128), neg.reshape(128, 128))
    return loss.reshape(())
